# baseline (speedup 1.0000x reference)
import functools

import jax
import jax.numpy as jnp
from jax import lax
from jax.experimental import pallas as pl
from jax.experimental.pallas import tpu as pltpu
from jax.experimental.pallas import tpu_sc as plsc

M, D = 131072, 128
TO_ADD_CONST = 1.0
CHUNK = 8192
NCHUNKS = M // CHUNK
NBUF = 6
B = 8
L = 16

_mesh = plsc.VectorSubcoreMesh(core_axis_name="c", subcore_axis_name="s")


@functools.partial(
    pl.kernel,
    out_type=jax.ShapeDtypeStruct((B, D), jnp.float32),
    mesh=_mesh,
    scratch_types=(
        pltpu.VMEM((B,), jnp.int32),
        pltpu.VMEM((B, D), jnp.float32),
        pltpu.SemaphoreType.DMA,
    ),
)
def _sc_gather_add(x_hbm, idxv_hbm, rows_out_hbm, idx_v, rows_v, sem):
    c = lax.axis_index("c")
    s = lax.axis_index("s")

    @pl.when((c == 0) & (s == 0))
    def _():
        pltpu.sync_copy(idxv_hbm, idx_v)
        pltpu.async_copy(x_hbm.at[idx_v], rows_v, sem).wait()
        for i in range(B):
            for j in range(D // L):
                sl = (i, pl.ds(j * L, L))
                rows_v[sl] = rows_v[sl] + TO_ADD_CONST
        pltpu.sync_copy(rows_v, rows_out_hbm)


def _tc_body(idx_ref, x_hbm, rows_ref, o_hbm, *rest):
    bufs = rest[:NBUF]
    in_sems = rest[NBUF:2 * NBUF]
    out_sems = rest[2 * NBUF:3 * NBUF]
    idx = idx_ref[0]

    def in_cp(k):
        b = k % NBUF
        return pltpu.make_async_copy(
            x_hbm.at[pl.ds(k * CHUNK, CHUNK), :], bufs[b], in_sems[b])

    def out_cp(k):
        b = k % NBUF
        return pltpu.make_async_copy(
            bufs[b], o_hbm.at[pl.ds(k * CHUNK, CHUNK), :], out_sems[b])

    for j in range(NBUF):
        in_cp(j).start()

    for k in range(NCHUNKS):
        in_cp(k).wait()
        b = k % NBUF
        base = k * CHUNK

        @pl.when((idx >= base) & (idx < base + CHUNK))
        def _(b=b, base=base):
            r = idx - base
            bufs[b][pl.ds(r, 1), :] = rows_ref[pl.ds(0, 1), :]

        out_cp(k).start()
        j = k + NBUF
        if j < NCHUNKS:
            out_cp(k).wait()
            in_cp(j).start()

    for k in range(max(NCHUNKS - NBUF, 0), NCHUNKS):
        out_cp(k).wait()


def _tc_copy_insert(idx_arr, rows, x):
    grid_spec = pltpu.PrefetchScalarGridSpec(
        num_scalar_prefetch=1,
        grid=(1,),
        in_specs=[
            pl.BlockSpec(memory_space=pl.ANY),
            pl.BlockSpec((B, D), lambda i, idx: (0, 0)),
        ],
        out_specs=pl.BlockSpec(memory_space=pl.ANY),
        scratch_shapes=(
            [pltpu.VMEM((CHUNK, D), jnp.float32)] * NBUF
            + [pltpu.SemaphoreType.DMA] * (2 * NBUF)
        ),
    )
    return pl.pallas_call(
        _tc_body,
        grid_spec=grid_spec,
        out_shape=jax.ShapeDtypeStruct((M, D), jnp.float32),
    )(idx_arr, x, rows)


def kernel(tensor, slice_index, related_index):
    idx_arr = jnp.asarray(slice_index, dtype=jnp.int32).reshape((1,))
    idxv = jnp.full((B,), slice_index, dtype=jnp.int32)
    rows = _sc_gather_add(tensor, idxv)
    out = _tc_copy_insert(idx_arr, rows, tensor)
    return (out, slice_index, related_index)
